# baseline (speedup 1.0000x reference)
import functools

import jax
import jax.numpy as jnp
from jax import lax
from jax.experimental import pallas as pl
from jax.experimental.pallas import tpu as pltpu
from jax.experimental.pallas import tpu_sc as plsc

EMB = 100
NC, NS, L = 2, 16, 16
NW = NC * NS
CHUNK = 128

_LOG1P = (
    9.083786844943376e-08, 0.9999914545717464, -0.49980116320372914,
    0.3313340057250358, -0.23919071732133323, 0.16478349729867933,
    -0.09231376866991943, 0.03441859352056854, -0.006074877643740236,
)


def _log_sigmoid(x):
    z = jnp.exp(-jnp.abs(x))
    p = jnp.full_like(z, _LOG1P[-1])
    for c in reversed(_LOG1P[:-1]):
        p = p * z + c
    return jnp.minimum(x, 0.0) - p


def _make_sc_kernel(batch, bpw):
    nchunk = bpw // CHUNK
    mesh = plsc.VectorSubcoreMesh(core_axis_name="c", subcore_axis_name="s")

    @functools.partial(
        pl.kernel,
        out_type=jax.ShapeDtypeStruct((batch,), jnp.float32),
        mesh=mesh,
        compiler_params=pltpu.CompilerParams(
            needs_layout_passes=False, use_tc_tiling_on_sc=False),
        scratch_types=[
            pltpu.VMEM((nchunk, CHUNK), jnp.int32),
            pltpu.VMEM((nchunk, CHUNK), jnp.int32),
            pltpu.VMEM((bpw, EMB), jnp.float32),
            pltpu.VMEM((bpw, EMB), jnp.float32),
            pltpu.VMEM((bpw,), jnp.float32),
            pltpu.SemaphoreType.DMA,
            pltpu.SemaphoreType.DMA,
        ],
    )
    def sc_kernel(inp_hbm, ctx_hbm, table_hbm, out_hbm,
                  iidx, cidx, irows, crows, outv, sem_i, sem_c):
        wid = lax.axis_index("s") * NC + lax.axis_index("c")
        pltpu.sync_copy(inp_hbm.at[wid], iidx)
        pltpu.sync_copy(ctx_hbm.at[wid], cidx)
        copies = []
        for ch in range(nchunk):
            dst = pl.ds(ch * CHUNK, CHUNK)
            copies.append(pltpu.async_copy(
                table_hbm.at[iidx.at[ch]], irows.at[dst], sem_i))
            copies.append(pltpu.async_copy(
                table_hbm.at[cidx.at[ch]], crows.at[dst], sem_c))
        for cp in copies:
            cp.wait()

        lane = lax.iota(jnp.int32, L)

        def group_body(g, carry):
            rows = g * L + lane
            accs = [jnp.zeros((L,), jnp.float32) for _ in range(4)]
            for d in range(EMB):
                col = jnp.full((L,), d, jnp.int32)
                a = plsc.load_gather(irows, [rows, col])
                b = plsc.load_gather(crows, [rows, col])
                accs[d % 4] = accs[d % 4] + a * b
            score = (accs[0] + accs[1]) + (accs[2] + accs[3])
            outv[pl.ds(g * L, L)] = _log_sigmoid(score)
            return carry

        lax.fori_loop(0, bpw // L, group_body, 0)
        pltpu.sync_copy(outv, out_hbm.at[pl.ds(wid * bpw, bpw)])

    return sc_kernel


def kernel(input, context, table):
    batch = input.shape[0]
    bpw = batch // NW
    inp = input.astype(jnp.int32).reshape(NW, bpw // CHUNK, CHUNK)
    ctx = context.astype(jnp.int32).reshape(NW, bpw // CHUNK, CHUNK)
    scores = _make_sc_kernel(batch, bpw)(inp, ctx, table)
    return scores.reshape(batch, 1)


# speedup vs baseline: 1.0936x; 1.0936x over previous
import functools

import jax
import jax.numpy as jnp
from jax import lax
from jax.experimental import pallas as pl
from jax.experimental.pallas import tpu as pltpu
from jax.experimental.pallas import tpu_sc as plsc

EMB = 100
PAD = 128
NC, NS, L = 2, 16, 16
NW = NC * NS
CHUNK = 128

_LOG1P = (
    9.083786844943376e-08, 0.9999914545717464, -0.49980116320372914,
    0.3313340057250358, -0.23919071732133323, 0.16478349729867933,
    -0.09231376866991943, 0.03441859352056854, -0.006074877643740236,
)


def _log_sigmoid(x):
    z = jnp.exp(-jnp.abs(x))
    p = jnp.full_like(z, _LOG1P[-1])
    for c in reversed(_LOG1P[:-1]):
        p = p * z + c
    return jnp.minimum(x, 0.0) - p


def _make_sc_kernel(batch, bpw):
    nchunk = bpw // CHUNK
    mesh = plsc.VectorSubcoreMesh(core_axis_name="c", subcore_axis_name="s")

    @functools.partial(
        pl.kernel,
        out_type=jax.ShapeDtypeStruct((batch,), jnp.float32),
        mesh=mesh,
        compiler_params=pltpu.CompilerParams(
            needs_layout_passes=False, use_tc_tiling_on_sc=False),
        scratch_types=[
            pltpu.VMEM((bpw,), jnp.int32),
            pltpu.VMEM((bpw,), jnp.int32),
            pltpu.VMEM((CHUNK, PAD), jnp.float32),
            pltpu.VMEM((CHUNK, PAD), jnp.float32),
            pltpu.VMEM((CHUNK, PAD), jnp.float32),
            pltpu.VMEM((CHUNK, PAD), jnp.float32),
            pltpu.VMEM((bpw,), jnp.float32),
            pltpu.SemaphoreType.DMA,
            pltpu.SemaphoreType.DMA,
        ],
    )
    def sc_kernel(inp_hbm, ctx_hbm, table_hbm, out_hbm,
                  iidx, cidx, ib0, ib1, cb0, cb1, outv, sem0, sem1):
        wid = lax.axis_index("s") * NC + lax.axis_index("c")
        pltpu.sync_copy(inp_hbm.at[wid], iidx)
        pltpu.sync_copy(ctx_hbm.at[wid], cidx)
        ibufs, cbufs, sems = (ib0, ib1), (cb0, cb1), (sem0, sem1)

        def start(ch):
            sl = ch % 2
            src = pl.ds(ch * CHUNK, CHUNK)
            return (
                pltpu.async_copy(table_hbm.at[iidx.at[src]], ibufs[sl], sems[sl]),
                pltpu.async_copy(table_hbm.at[cidx.at[src]], cbufs[sl], sems[sl]),
            )

        lane = lax.iota(jnp.int32, L)
        pending = start(0)
        for ch in range(nchunk):
            nxt = start(ch + 1) if ch + 1 < nchunk else None
            for cp in pending:
                cp.wait()
            ib, cb = ibufs[ch % 2], cbufs[ch % 2]

            def group_body(g, carry, ib=ib, cb=cb, ch=ch):
                rows = g * L + lane
                accs = [jnp.zeros((L,), jnp.float32) for _ in range(4)]
                for d in range(EMB):
                    col = jnp.full((L,), d, jnp.int32)
                    a = plsc.load_gather(ib, [rows, col])
                    b = plsc.load_gather(cb, [rows, col])
                    accs[d % 4] = accs[d % 4] + a * b
                score = (accs[0] + accs[1]) + (accs[2] + accs[3])
                outv[pl.ds(ch * CHUNK + g * L, L)] = _log_sigmoid(score)
                return carry

            lax.fori_loop(0, CHUNK // L, group_body, 0)
            pending = nxt

        pltpu.sync_copy(outv, out_hbm.at[pl.ds(wid * bpw, bpw)])

    return sc_kernel


def kernel(input, context, table):
    batch = input.shape[0]
    bpw = batch // NW
    inp = input.astype(jnp.int32).reshape(NW, bpw)
    ctx = context.astype(jnp.int32).reshape(NW, bpw)
    tablep = jnp.pad(table, ((0, 0), (0, PAD - EMB)))
    scores = _make_sc_kernel(batch, bpw)(inp, ctx, tablep)
    return scores.reshape(batch, 1)


# speedup vs baseline: 1.1848x; 1.0834x over previous
import functools

import jax
import jax.numpy as jnp
from jax import lax
from jax.experimental import pallas as pl
from jax.experimental.pallas import tpu as pltpu
from jax.experimental.pallas import tpu_sc as plsc

EMB = 100
PAD = 128
NC, NS, L = 2, 16, 16
NW = NC * NS
CHUNK = 128

_LOG1P = (
    9.083786844943376e-08, 0.9999914545717464, -0.49980116320372914,
    0.3313340057250358, -0.23919071732133323, 0.16478349729867933,
    -0.09231376866991943, 0.03441859352056854, -0.006074877643740236,
)


def _log_sigmoid(x):
    z = jnp.exp(-jnp.abs(x))
    p = jnp.full_like(z, _LOG1P[-1])
    for c in reversed(_LOG1P[:-1]):
        p = p * z + c
    return jnp.minimum(x, 0.0) - p


def _make_sc_kernel(batch, bpw):
    nchunk = bpw // CHUNK
    mesh = plsc.VectorSubcoreMesh(core_axis_name="c", subcore_axis_name="s")

    @functools.partial(
        pl.kernel,
        out_type=jax.ShapeDtypeStruct((batch,), jnp.float32),
        mesh=mesh,
        compiler_params=pltpu.CompilerParams(needs_layout_passes=False),
        scratch_types=[
            pltpu.VMEM((bpw,), jnp.int32),
            pltpu.VMEM((bpw,), jnp.int32),
            pltpu.VMEM((CHUNK, PAD), jnp.float32),
            pltpu.VMEM((CHUNK, PAD), jnp.float32),
            pltpu.VMEM((CHUNK, PAD), jnp.float32),
            pltpu.VMEM((CHUNK, PAD), jnp.float32),
            pltpu.VMEM((bpw,), jnp.float32),
            pltpu.SemaphoreType.DMA,
            pltpu.SemaphoreType.DMA,
        ],
    )
    def sc_kernel(inp_hbm, ctx_hbm, table_hbm, out_hbm,
                  iidx, cidx, ib0, ib1, cb0, cb1, outv, sem0, sem1):
        wid = lax.axis_index("s") * NC + lax.axis_index("c")
        pltpu.sync_copy(inp_hbm.at[wid], iidx)
        pltpu.sync_copy(ctx_hbm.at[wid], cidx)
        ibufs, cbufs, sems = (ib0, ib1), (cb0, cb1), (sem0, sem1)

        def start(ch):
            sl = ch % 2
            src = pl.ds(ch * CHUNK, CHUNK)
            return (
                pltpu.async_copy(table_hbm.at[iidx.at[src]], ibufs[sl], sems[sl]),
                pltpu.async_copy(table_hbm.at[cidx.at[src]], cbufs[sl], sems[sl]),
            )

        lane = lax.iota(jnp.int32, L)
        pending = start(0)
        for ch in range(nchunk):
            nxt = start(ch + 1) if ch + 1 < nchunk else None
            for cp in pending:
                cp.wait()
            ib, cb = ibufs[ch % 2], cbufs[ch % 2]

            def group_body(g, carry, ib=ib, cb=cb, ch=ch):
                rows = g * L + lane
                col = lane
                accs = [jnp.zeros((L,), jnp.float32) for _ in range(4)]
                for t in range(EMB):
                    a = plsc.load_gather(ib, [rows, col])
                    b = plsc.load_gather(cb, [rows, col])
                    accs[t % 4] = accs[t % 4] + a * b
                    col = col + 1
                score = (accs[0] + accs[1]) + (accs[2] + accs[3])
                outv[pl.ds(ch * CHUNK + g * L, L)] = _log_sigmoid(score)
                return carry

            lax.fori_loop(0, CHUNK // L, group_body, 0)
            pending = nxt

        pltpu.sync_copy(outv, out_hbm.at[pl.ds(wid * bpw, bpw)])

    return sc_kernel


def kernel(input, context, table):
    batch = input.shape[0]
    bpw = batch // NW
    inp = input.astype(jnp.int32).reshape(NW, bpw)
    ctx = context.astype(jnp.int32).reshape(NW, bpw)
    tablep = jnp.concatenate([table, table[:, :PAD - EMB]], axis=1)
    scores = _make_sc_kernel(batch, bpw)(inp, ctx, tablep)
    return scores.reshape(batch, 1)


# speedup vs baseline: 2.7015x; 2.2801x over previous
import functools

import jax
import jax.numpy as jnp
from jax import lax
from jax.experimental import pallas as pl
from jax.experimental.pallas import tpu as pltpu
from jax.experimental.pallas import tpu_sc as plsc

EMB = 100
PAD = 128
NC, NS, L = 2, 16, 16
NW = NC * NS
CHUNK = 128

_LOG1P = (
    9.083786844943376e-08, 0.9999914545717464, -0.49980116320372914,
    0.3313340057250358, -0.23919071732133323, 0.16478349729867933,
    -0.09231376866991943, 0.03441859352056854, -0.006074877643740236,
)


def _log_sigmoid(x):
    z = jnp.exp(-jnp.abs(x))
    p = jnp.full_like(z, _LOG1P[-1])
    for c in reversed(_LOG1P[:-1]):
        p = p * z + c
    return jnp.minimum(x, 0.0) - p


def _widen_table(table):
    vocab = table.shape[0]
    tableT = table.T
    eye = jnp.eye(EMB, dtype=jnp.float32)
    sel = jnp.concatenate([eye, eye[:, :PAD - EMB]], axis=1)
    vb = 2048

    def body(t_ref, e_ref, o_ref):
        o_ref[...] = jax.lax.dot_general(
            t_ref[...], e_ref[...], (((0,), (0,)), ((), ())),
            preferred_element_type=jnp.float32,
            precision=jax.lax.Precision.HIGHEST)

    return pl.pallas_call(
        body,
        grid=(pl.cdiv(vocab, vb),),
        in_specs=[
            pl.BlockSpec((EMB, vb), lambda i: (0, i)),
            pl.BlockSpec((EMB, PAD), lambda i: (0, 0)),
        ],
        out_specs=pl.BlockSpec((vb, PAD), lambda i: (i, 0)),
        out_shape=jax.ShapeDtypeStruct((vocab, PAD), jnp.float32),
    )(tableT, sel)


def _make_sc_kernel(batch, bpw):
    nchunk = bpw // CHUNK
    mesh = plsc.VectorSubcoreMesh(core_axis_name="c", subcore_axis_name="s")

    @functools.partial(
        pl.kernel,
        out_type=jax.ShapeDtypeStruct((batch,), jnp.float32),
        mesh=mesh,
        compiler_params=pltpu.CompilerParams(needs_layout_passes=False),
        scratch_types=[
            pltpu.VMEM((bpw,), jnp.int32),
            pltpu.VMEM((bpw,), jnp.int32),
            pltpu.VMEM((CHUNK, PAD), jnp.float32),
            pltpu.VMEM((CHUNK, PAD), jnp.float32),
            pltpu.VMEM((CHUNK, PAD), jnp.float32),
            pltpu.VMEM((CHUNK, PAD), jnp.float32),
            pltpu.VMEM((bpw,), jnp.float32),
            pltpu.SemaphoreType.DMA,
            pltpu.SemaphoreType.DMA,
        ],
    )
    def sc_kernel(inp_hbm, ctx_hbm, table_hbm, out_hbm,
                  iidx, cidx, ib0, ib1, cb0, cb1, outv, sem0, sem1):
        wid = lax.axis_index("s") * NC + lax.axis_index("c")
        pltpu.sync_copy(inp_hbm.at[wid], iidx)
        pltpu.sync_copy(ctx_hbm.at[wid], cidx)
        ibufs, cbufs, sems = (ib0, ib1), (cb0, cb1), (sem0, sem1)

        def start(ch):
            sl = ch % 2
            src = pl.ds(ch * CHUNK, CHUNK)
            return (
                pltpu.async_copy(table_hbm.at[iidx.at[src]], ibufs[sl], sems[sl]),
                pltpu.async_copy(table_hbm.at[cidx.at[src]], cbufs[sl], sems[sl]),
            )

        lane = lax.iota(jnp.int32, L)
        pending = start(0)
        for ch in range(nchunk):
            nxt = start(ch + 1) if ch + 1 < nchunk else None
            for cp in pending:
                cp.wait()
            ib, cb = ibufs[ch % 2], cbufs[ch % 2]

            def group_body(g, carry, ib=ib, cb=cb, ch=ch):
                rows = g * L + lane
                col = lane
                accs = [jnp.zeros((L,), jnp.float32) for _ in range(4)]
                for t in range(EMB):
                    a = plsc.load_gather(ib, [rows, col])
                    b = plsc.load_gather(cb, [rows, col])
                    accs[t % 4] = accs[t % 4] + a * b
                    col = col + 1
                score = (accs[0] + accs[1]) + (accs[2] + accs[3])
                outv[pl.ds(ch * CHUNK + g * L, L)] = _log_sigmoid(score)
                return carry

            lax.fori_loop(0, CHUNK // L, group_body, 0)
            pending = nxt

        pltpu.sync_copy(outv, out_hbm.at[pl.ds(wid * bpw, bpw)])

    return sc_kernel


def kernel(input, context, table):
    batch = input.shape[0]
    bpw = batch // NW
    inp = input.astype(jnp.int32).reshape(NW, bpw)
    ctx = context.astype(jnp.int32).reshape(NW, bpw)
    tablep = _widen_table(table)
    scores = _make_sc_kernel(batch, bpw)(inp, ctx, tablep)
    return scores.reshape(batch, 1)


# speedup vs baseline: 3.0950x; 1.1456x over previous
import functools

import jax
import jax.numpy as jnp
from jax import lax
from jax.experimental import pallas as pl
from jax.experimental.pallas import tpu as pltpu
from jax.experimental.pallas import tpu_sc as plsc

EMB = 100
PAD = 128
NC, NS, L = 2, 16, 16
NW = NC * NS
CHUNK = 128

_LOG1P = (
    9.083786844943376e-08, 0.9999914545717464, -0.49980116320372914,
    0.3313340057250358, -0.23919071732133323, 0.16478349729867933,
    -0.09231376866991943, 0.03441859352056854, -0.006074877643740236,
)


def _log_sigmoid(x):
    z = jnp.exp(-jnp.abs(x))
    p = jnp.full_like(z, _LOG1P[-1])
    for c in reversed(_LOG1P[:-1]):
        p = p * z + c
    return jnp.minimum(x, 0.0) - p


def _widen_table(table):
    vocab = table.shape[0]
    tableT = table.T
    eye = jnp.eye(EMB, dtype=jnp.float32)
    sel = jnp.concatenate([eye, eye[:, :PAD - EMB]], axis=1)
    vb = 2048

    def body(t_ref, e_ref, o_ref):
        o_ref[...] = jax.lax.dot_general(
            t_ref[...], e_ref[...], (((0,), (0,)), ((), ())),
            preferred_element_type=jnp.float32,
            precision=jax.lax.Precision.HIGHEST)

    return pl.pallas_call(
        body,
        grid=(pl.cdiv(vocab, vb),),
        in_specs=[
            pl.BlockSpec((EMB, vb), lambda i: (0, i)),
            pl.BlockSpec((EMB, PAD), lambda i: (0, 0)),
        ],
        out_specs=pl.BlockSpec((vb, PAD), lambda i: (i, 0)),
        out_shape=jax.ShapeDtypeStruct((vocab, PAD), jnp.float32),
    )(tableT, sel)


def _make_sc_kernel(batch, bpw):
    nchunk = bpw // CHUNK
    mesh = plsc.VectorSubcoreMesh(core_axis_name="c", subcore_axis_name="s")

    @functools.partial(
        pl.kernel,
        out_type=jax.ShapeDtypeStruct((batch,), jnp.float32),
        mesh=mesh,
        compiler_params=pltpu.CompilerParams(needs_layout_passes=False),
        scratch_types=[
            pltpu.VMEM((bpw,), jnp.int32),
            pltpu.VMEM((bpw,), jnp.int32),
            pltpu.VMEM((CHUNK, PAD), jnp.float32),
            pltpu.VMEM((CHUNK, PAD), jnp.float32),
            pltpu.VMEM((CHUNK, PAD), jnp.float32),
            pltpu.VMEM((CHUNK, PAD), jnp.float32),
            pltpu.VMEM((bpw,), jnp.float32),
            pltpu.SemaphoreType.DMA,
            pltpu.SemaphoreType.DMA,
        ],
    )
    def sc_kernel(inp_hbm, ctx_hbm, table_hbm, out_hbm,
                  iidx, cidx, ib0, ib1, cb0, cb1, outv, sem0, sem1):
        wid = lax.axis_index("s") * NC + lax.axis_index("c")
        pltpu.sync_copy(inp_hbm.at[wid], iidx)
        pltpu.sync_copy(ctx_hbm.at[wid], cidx)
        ibufs, cbufs, sems = (ib0, ib1), (cb0, cb1), (sem0, sem1)

        def start(ch):
            sl = ch % 2
            src = pl.ds(ch * CHUNK, CHUNK)
            return (
                pltpu.async_copy(table_hbm.at[iidx.at[src]], ibufs[sl], sems[sl]),
                pltpu.async_copy(table_hbm.at[cidx.at[src]], cbufs[sl], sems[sl]),
            )

        lane = lax.iota(jnp.int32, L)
        tb = 20
        zero = jnp.zeros((L,), jnp.float32)
        pending = start(0)
        for ch in range(nchunk):
            nxt = start(ch + 1) if ch + 1 < nchunk else None
            for cp in pending:
                cp.wait()
            ib, cb = ibufs[ch % 2], cbufs[ch % 2]

            @plsc.parallel_loop(0, CHUNK // L)
            def _group(g, ib=ib, cb=cb, ch=ch):
                rows = g * L + lane
                def tb_body(j, accs, ib=ib, cb=cb, rows=rows):
                    colb = j * tb + lane
                    for k in range(tb):
                        col = colb + k
                        a = plsc.load_gather(ib, [rows, col])
                        b = plsc.load_gather(cb, [rows, col])
                        accs = (accs[1], accs[2], accs[3], accs[0] + a * b)
                    return accs

                accs = lax.fori_loop(0, EMB // tb, tb_body,
                                     (zero, zero, zero, zero))
                score = (accs[0] + accs[1]) + (accs[2] + accs[3])
                outv[pl.ds(ch * CHUNK + g * L, L)] = _log_sigmoid(score)

            pending = nxt

        pltpu.sync_copy(outv, out_hbm.at[pl.ds(wid * bpw, bpw)])

    return sc_kernel


def kernel(input, context, table):
    batch = input.shape[0]
    bpw = batch // NW
    inp = input.astype(jnp.int32).reshape(NW, bpw)
    ctx = context.astype(jnp.int32).reshape(NW, bpw)
    tablep = _widen_table(table)
    scores = _make_sc_kernel(batch, bpw)(inp, ctx, tablep)
    return scores.reshape(batch, 1)


# speedup vs baseline: 3.5873x; 1.1591x over previous
import functools

import jax
import jax.numpy as jnp
from jax import lax
from jax.experimental import pallas as pl
from jax.experimental.pallas import tpu as pltpu
from jax.experimental.pallas import tpu_sc as plsc

EMB = 100
PAD = 128
NC, NS, L = 2, 16, 16
NW = NC * NS
CHUNK = 128

_LOG1P = (
    9.083786844943376e-08, 0.9999914545717464, -0.49980116320372914,
    0.3313340057250358, -0.23919071732133323, 0.16478349729867933,
    -0.09231376866991943, 0.03441859352056854, -0.006074877643740236,
)


def _log_sigmoid(x):
    z = jnp.exp(-jnp.abs(x))
    p = jnp.full_like(z, _LOG1P[-1])
    for c in reversed(_LOG1P[:-1]):
        p = p * z + c
    return jnp.minimum(x, 0.0) - p


def _widen_table(table):
    vocab = table.shape[0]
    tableT = table.T
    eye = jnp.eye(EMB, dtype=jnp.float32)
    sel = jnp.concatenate([eye, eye[:, :PAD - EMB]], axis=1)
    vb = 4096

    def body(t_ref, e_ref, o_ref):
        o_ref[...] = jax.lax.dot_general(
            t_ref[...], e_ref[...], (((0,), (0,)), ((), ())),
            preferred_element_type=jnp.float32,
            precision=jax.lax.Precision.HIGHEST)

    return pl.pallas_call(
        body,
        grid=(pl.cdiv(vocab, vb),),
        in_specs=[
            pl.BlockSpec((EMB, vb), lambda i: (0, i)),
            pl.BlockSpec((EMB, PAD), lambda i: (0, 0)),
        ],
        out_specs=pl.BlockSpec((vb, PAD), lambda i: (i, 0)),
        out_shape=jax.ShapeDtypeStruct((vocab, PAD), jnp.float32),
    )(tableT, sel)


def _make_sc_kernel(batch, bpw):
    nchunk = bpw // CHUNK
    mesh = plsc.VectorSubcoreMesh(core_axis_name="c", subcore_axis_name="s")

    @functools.partial(
        pl.kernel,
        out_type=jax.ShapeDtypeStruct((batch,), jnp.float32),
        mesh=mesh,
        compiler_params=pltpu.CompilerParams(needs_layout_passes=False),
        scratch_types=[
            pltpu.VMEM((bpw,), jnp.int32),
            pltpu.VMEM((bpw,), jnp.int32),
            pltpu.VMEM((CHUNK, PAD), jnp.float32),
            pltpu.VMEM((CHUNK, PAD), jnp.float32),
            pltpu.VMEM((CHUNK, PAD), jnp.float32),
            pltpu.VMEM((CHUNK, PAD), jnp.float32),
            pltpu.VMEM((bpw,), jnp.float32),
            pltpu.SemaphoreType.DMA,
            pltpu.SemaphoreType.DMA,
        ],
    )
    def sc_kernel(inp_hbm, ctx_hbm, table_hbm, out_hbm,
                  iidx, cidx, ib0, ib1, cb0, cb1, outv, sem0, sem1):
        wid = lax.axis_index("s") * NC + lax.axis_index("c")
        pltpu.sync_copy(inp_hbm.at[wid], iidx)
        pltpu.sync_copy(ctx_hbm.at[wid], cidx)
        ibufs, cbufs, sems = (ib0, ib1), (cb0, cb1), (sem0, sem1)

        def start(ch):
            sl = ch % 2
            src = pl.ds(ch * CHUNK, CHUNK)
            return (
                pltpu.async_copy(table_hbm.at[iidx.at[src]], ibufs[sl], sems[sl]),
                pltpu.async_copy(table_hbm.at[cidx.at[src]], cbufs[sl], sems[sl]),
            )

        lane = lax.iota(jnp.int32, L)
        tb = 20
        zero = jnp.zeros((L,), jnp.float32)
        pending = start(0)
        for ch in range(nchunk):
            nxt = start(ch + 1) if ch + 1 < nchunk else None
            for cp in pending:
                cp.wait()
            ib, cb = ibufs[ch % 2], cbufs[ch % 2]

            @plsc.parallel_loop(0, CHUNK // L)
            def _group(g, ib=ib, cb=cb, ch=ch):
                rows = g * L + lane
                def tb_body(j, accs, ib=ib, cb=cb, rows=rows):
                    colb = j * tb + lane
                    for k in range(tb):
                        col = colb + k
                        a = plsc.load_gather(ib, [rows, col])
                        b = plsc.load_gather(cb, [rows, col])
                        accs = (accs[1], accs[2], accs[3], accs[0] + a * b)
                    return accs

                accs = lax.fori_loop(0, EMB // tb, tb_body,
                                     (zero, zero, zero, zero))
                score = (accs[0] + accs[1]) + (accs[2] + accs[3])
                outv[pl.ds(ch * CHUNK + g * L, L)] = _log_sigmoid(score)

            pending = nxt

        pltpu.sync_copy(outv, out_hbm.at[pl.ds(wid * bpw, bpw)])

    return sc_kernel


def kernel(input, context, table):
    batch = input.shape[0]
    bpw = batch // NW
    inp = input.astype(jnp.int32).reshape(NW, bpw)
    ctx = context.astype(jnp.int32).reshape(NW, bpw)
    tablep = _widen_table(table)
    scores = _make_sc_kernel(batch, bpw)(inp, ctx, tablep)
    return scores.reshape(batch, 1)


# speedup vs baseline: 3.7987x; 1.0589x over previous
import functools

import jax
import jax.numpy as jnp
from jax import lax
from jax.experimental import pallas as pl
from jax.experimental.pallas import tpu as pltpu
from jax.experimental.pallas import tpu_sc as plsc

EMB = 100
PAD = 128
NC, NS, L = 2, 16, 16
NW = NC * NS
CHUNK = 128

_LOG1P = (
    9.083786844943376e-08, 0.9999914545717464, -0.49980116320372914,
    0.3313340057250358, -0.23919071732133323, 0.16478349729867933,
    -0.09231376866991943, 0.03441859352056854, -0.006074877643740236,
)


def _log_sigmoid(x):
    z = jnp.exp(-jnp.abs(x))
    p = jnp.full_like(z, _LOG1P[-1])
    for c in reversed(_LOG1P[:-1]):
        p = p * z + c
    return jnp.minimum(x, 0.0) - p


def _widen_table(table):
    vocab = table.shape[0]
    tableT = table.T
    eye = jnp.eye(EMB, dtype=jnp.float32)
    sel = jnp.concatenate([eye, eye[:, :PAD - EMB]], axis=1)
    vb = 8192

    def body(t_ref, e_ref, o_ref):
        o_ref[...] = jax.lax.dot_general(
            t_ref[...], e_ref[...], (((0,), (0,)), ((), ())),
            preferred_element_type=jnp.float32,
            precision=jax.lax.Precision.HIGHEST)

    return pl.pallas_call(
        body,
        grid=(pl.cdiv(vocab, vb),),
        in_specs=[
            pl.BlockSpec((EMB, vb), lambda i: (0, i)),
            pl.BlockSpec((EMB, PAD), lambda i: (0, 0)),
        ],
        out_specs=pl.BlockSpec((vb, PAD), lambda i: (i, 0)),
        out_shape=jax.ShapeDtypeStruct((vocab, PAD), jnp.float32),
    )(tableT, sel)


def _make_sc_kernel(batch, bpw):
    nchunk = bpw // CHUNK
    mesh = plsc.VectorSubcoreMesh(core_axis_name="c", subcore_axis_name="s")

    @functools.partial(
        pl.kernel,
        out_type=jax.ShapeDtypeStruct((batch,), jnp.float32),
        mesh=mesh,
        compiler_params=pltpu.CompilerParams(needs_layout_passes=False),
        scratch_types=[
            pltpu.VMEM((bpw,), jnp.int32),
            pltpu.VMEM((bpw,), jnp.int32),
            pltpu.VMEM((CHUNK, PAD), jnp.float32),
            pltpu.VMEM((CHUNK, PAD), jnp.float32),
            pltpu.VMEM((CHUNK, PAD), jnp.float32),
            pltpu.VMEM((CHUNK, PAD), jnp.float32),
            pltpu.VMEM((bpw,), jnp.float32),
            pltpu.SemaphoreType.DMA,
            pltpu.SemaphoreType.DMA,
        ],
    )
    def sc_kernel(inp_hbm, ctx_hbm, table_hbm, out_hbm,
                  iidx, cidx, ib0, ib1, cb0, cb1, outv, sem0, sem1):
        wid = lax.axis_index("s") * NC + lax.axis_index("c")
        pltpu.sync_copy(inp_hbm.at[wid], iidx)
        pltpu.sync_copy(ctx_hbm.at[wid], cidx)
        ibufs, cbufs, sems = (ib0, ib1), (cb0, cb1), (sem0, sem1)

        def start(ch):
            sl = ch % 2
            src = pl.ds(ch * CHUNK, CHUNK)
            return (
                pltpu.async_copy(table_hbm.at[iidx.at[src]], ibufs[sl], sems[sl]),
                pltpu.async_copy(table_hbm.at[cidx.at[src]], cbufs[sl], sems[sl]),
            )

        lane = lax.iota(jnp.int32, L)
        tb = 20
        zero = jnp.zeros((L,), jnp.float32)
        pending = start(0)
        for ch in range(nchunk):
            nxt = start(ch + 1) if ch + 1 < nchunk else None
            for cp in pending:
                cp.wait()
            ib, cb = ibufs[ch % 2], cbufs[ch % 2]

            @plsc.parallel_loop(0, CHUNK // L)
            def _group(g, ib=ib, cb=cb, ch=ch):
                rows = g * L + lane
                def tb_body(j, accs, ib=ib, cb=cb, rows=rows):
                    colb = j * tb + lane
                    for k in range(tb):
                        col = colb + k
                        a = plsc.load_gather(ib, [rows, col])
                        b = plsc.load_gather(cb, [rows, col])
                        accs = (accs[1], accs[2], accs[3], accs[0] + a * b)
                    return accs

                accs = lax.fori_loop(0, EMB // tb, tb_body,
                                     (zero, zero, zero, zero))
                score = (accs[0] + accs[1]) + (accs[2] + accs[3])
                outv[pl.ds(ch * CHUNK + g * L, L)] = _log_sigmoid(score)

            pending = nxt

        pltpu.sync_copy(outv, out_hbm.at[pl.ds(wid * bpw, bpw)])

    return sc_kernel


def kernel(input, context, table):
    batch = input.shape[0]
    bpw = batch // NW
    inp = input.astype(jnp.int32).reshape(NW, bpw)
    ctx = context.astype(jnp.int32).reshape(NW, bpw)
    tablep = _widen_table(table)
    scores = _make_sc_kernel(batch, bpw)(inp, ctx, tablep)
    return scores.reshape(batch, 1)


# speedup vs baseline: 3.8441x; 1.0119x over previous
import functools

import jax
import jax.numpy as jnp
from jax import lax
from jax.experimental import pallas as pl
from jax.experimental.pallas import tpu as pltpu
from jax.experimental.pallas import tpu_sc as plsc

EMB = 100
PAD = 128
NC, NS, L = 2, 16, 16
NW = NC * NS
CHUNK = 128

_LOG1P = (
    9.083786844943376e-08, 0.9999914545717464, -0.49980116320372914,
    0.3313340057250358, -0.23919071732133323, 0.16478349729867933,
    -0.09231376866991943, 0.03441859352056854, -0.006074877643740236,
)


def _log_sigmoid(x):
    z = jnp.exp(-jnp.abs(x))
    p = jnp.full_like(z, _LOG1P[-1])
    for c in reversed(_LOG1P[:-1]):
        p = p * z + c
    return jnp.minimum(x, 0.0) - p


def _widen_table(table):
    vocab = table.shape[0]
    tableT = table.T
    eye = jnp.eye(EMB, dtype=jnp.float32)
    sel = jnp.concatenate([eye, eye[:, :PAD - EMB]], axis=1)
    vb = 12800

    def body(t_ref, e_ref, o_ref):
        o_ref[...] = jax.lax.dot_general(
            t_ref[...], e_ref[...], (((0,), (0,)), ((), ())),
            preferred_element_type=jnp.float32,
            precision=jax.lax.Precision.HIGHEST)

    return pl.pallas_call(
        body,
        grid=(pl.cdiv(vocab, vb),),
        in_specs=[
            pl.BlockSpec((EMB, vb), lambda i: (0, i)),
            pl.BlockSpec((EMB, PAD), lambda i: (0, 0)),
        ],
        out_specs=pl.BlockSpec((vb, PAD), lambda i: (i, 0)),
        out_shape=jax.ShapeDtypeStruct((vocab, PAD), jnp.float32),
    )(tableT, sel)


def _make_sc_kernel(batch, bpw):
    nchunk = bpw // CHUNK
    mesh = plsc.VectorSubcoreMesh(core_axis_name="c", subcore_axis_name="s")

    @functools.partial(
        pl.kernel,
        out_type=jax.ShapeDtypeStruct((batch,), jnp.float32),
        mesh=mesh,
        compiler_params=pltpu.CompilerParams(needs_layout_passes=False),
        scratch_types=[
            pltpu.VMEM((bpw,), jnp.int32),
            pltpu.VMEM((bpw,), jnp.int32),
            pltpu.VMEM((CHUNK, PAD), jnp.float32),
            pltpu.VMEM((CHUNK, PAD), jnp.float32),
            pltpu.VMEM((CHUNK, PAD), jnp.float32),
            pltpu.VMEM((CHUNK, PAD), jnp.float32),
            pltpu.VMEM((bpw,), jnp.float32),
            pltpu.SemaphoreType.DMA,
            pltpu.SemaphoreType.DMA,
        ],
    )
    def sc_kernel(inp_hbm, ctx_hbm, table_hbm, out_hbm,
                  iidx, cidx, ib0, ib1, cb0, cb1, outv, sem0, sem1):
        wid = lax.axis_index("s") * NC + lax.axis_index("c")
        pltpu.sync_copy(inp_hbm.at[wid], iidx)
        pltpu.sync_copy(ctx_hbm.at[wid], cidx)
        ibufs, cbufs, sems = (ib0, ib1), (cb0, cb1), (sem0, sem1)

        def start(ch):
            sl = ch % 2
            src = pl.ds(ch * CHUNK, CHUNK)
            return (
                pltpu.async_copy(table_hbm.at[iidx.at[src]], ibufs[sl], sems[sl]),
                pltpu.async_copy(table_hbm.at[cidx.at[src]], cbufs[sl], sems[sl]),
            )

        lane = lax.iota(jnp.int32, L)
        tb = 20
        zero = jnp.zeros((L,), jnp.float32)
        pending = start(0)
        for ch in range(nchunk):
            nxt = start(ch + 1) if ch + 1 < nchunk else None
            for cp in pending:
                cp.wait()
            ib, cb = ibufs[ch % 2], cbufs[ch % 2]

            @plsc.parallel_loop(0, CHUNK // L)
            def _group(g, ib=ib, cb=cb, ch=ch):
                rows = g * L + lane
                def tb_body(j, accs, ib=ib, cb=cb, rows=rows):
                    colb = j * tb + lane
                    for k in range(tb):
                        col = colb + k
                        a = plsc.load_gather(ib, [rows, col])
                        b = plsc.load_gather(cb, [rows, col])
                        accs = (accs[1], accs[2], accs[3], accs[0] + a * b)
                    return accs

                accs = lax.fori_loop(0, EMB // tb, tb_body,
                                     (zero, zero, zero, zero))
                score = (accs[0] + accs[1]) + (accs[2] + accs[3])
                outv[pl.ds(ch * CHUNK + g * L, L)] = _log_sigmoid(score)

            pending = nxt

        pltpu.sync_copy(outv, out_hbm.at[pl.ds(wid * bpw, bpw)])

    return sc_kernel


def kernel(input, context, table):
    batch = input.shape[0]
    bpw = batch // NW
    inp = input.astype(jnp.int32).reshape(NW, bpw)
    ctx = context.astype(jnp.int32).reshape(NW, bpw)
    tablep = _widen_table(table)
    scores = _make_sc_kernel(batch, bpw)(inp, ctx, tablep)
    return scores.reshape(batch, 1)
